# Initial kernel scaffold; baseline (speedup 1.0000x reference)
import jax
import jax.numpy as jnp
from jax.experimental import pallas as pl


def kernel(x, log_duration_prediction, max_len, duration_target):
    raise NotImplementedError("write your pallas kernel here")


# speedup vs baseline: 6.3110x; 6.3110x over previous
import functools

import jax
import jax.numpy as jnp
from jax import lax
from jax.experimental import pallas as pl
from jax.experimental.pallas import tpu as pltpu
from jax.experimental.pallas import tpu_sc as plsc

B, T, D = 8, 2048, 256
MAX_LEN = 14336
NC, NS, L = 2, 16, 16
BPC = B // NC
ROWS_PER_CORE = BPC * MAX_LEN
ROWS_PER_TILE = ROWS_PER_CORE // NS
CHUNK = 128
N_CHUNKS = ROWS_PER_TILE // CHUNK
ZERO_ROW = B * T
TCH = T // L
PCH = MAX_LEN // L


def _sc_body(table_hbm, dur_hbm, out_hbm, mel_hbm,
             dur_v, arr_v, gidx_sh, idx_v, rows_v, mel_v, sem):
    c = lax.axis_index("c")
    s = lax.axis_index("s")

    @pl.when(s < BPC)
    def _build_indices():
        b = c * BPC + s
        pltpu.sync_copy(dur_hbm.at[b], dur_v)

        def zero_body(j, carry):
            arr_v[pl.ds(j * L, L)] = jnp.zeros((L,), jnp.int32)
            return carry
        lax.fori_loop(0, PCH, zero_body, jnp.int32(0))

        def scatter_body(i, carry):
            dv = dur_v[pl.ds(i * L, L)]
            cs = plsc.cumsum(dv) + carry
            start = cs - dv
            vals = lax.iota(jnp.int32, L) + i * L
            ok = (dv > 0) & (start < MAX_LEN)
            plsc.store_scatter(arr_v, [start], vals, mask=ok)
            return jnp.max(cs)
        total = lax.fori_loop(0, TCH, scatter_body, jnp.int32(0))

        base = b * T

        def fill_body(j, carry):
            a = arr_v[pl.ds(j * L, L)]
            m = jnp.maximum(plsc.cummax(a), carry)
            pos = lax.iota(jnp.int32, L) + j * L
            arr_v[pl.ds(j * L, L)] = jnp.where(pos < total, base + m, ZERO_ROW)
            return jnp.max(m)
        lax.fori_loop(0, PCH, fill_body, jnp.int32(0))

        pltpu.sync_copy(arr_v, gidx_sh.at[pl.ds(s * MAX_LEN, MAX_LEN)])
        mel_v[...] = jnp.zeros((L,), jnp.int32) + total
        pltpu.sync_copy(mel_v, mel_hbm.at[b])

    plsc.subcore_barrier()

    core_row_base = c * ROWS_PER_CORE
    tile_off = s * ROWS_PER_TILE

    def gather_body(k, carry):
        off = tile_off + k * CHUNK
        pltpu.sync_copy(gidx_sh.at[pl.ds(off, CHUNK)], idx_v)
        pltpu.async_copy(table_hbm.at[idx_v], rows_v, sem).wait()
        pltpu.sync_copy(rows_v, out_hbm.at[pl.ds(core_row_base + off, CHUNK)])
        return carry
    lax.fori_loop(0, N_CHUNKS, gather_body, jnp.int32(0))


@jax.jit
def _upsample_sc(table, duration):
    mesh = plsc.VectorSubcoreMesh(core_axis_name="c", subcore_axis_name="s")
    f = pl.kernel(
        _sc_body,
        mesh=mesh,
        compiler_params=pltpu.CompilerParams(needs_layout_passes=False),
        out_type=[
            jax.ShapeDtypeStruct((B * MAX_LEN, D), jnp.float32),
            jax.ShapeDtypeStruct((B, L), jnp.int32),
        ],
        scratch_types=[
            pltpu.VMEM((T,), jnp.int32),
            pltpu.VMEM((MAX_LEN,), jnp.int32),
            pltpu.VMEM_SHARED((BPC * MAX_LEN,), jnp.int32),
            pltpu.VMEM((CHUNK,), jnp.int32),
            pltpu.VMEM((CHUNK, D), jnp.float32),
            pltpu.VMEM((L,), jnp.int32),
            pltpu.SemaphoreType.DMA,
        ],
    )
    return f(table, duration)


def kernel(x, log_duration_prediction, max_len, duration_target):
    del log_duration_prediction, max_len
    table = jnp.concatenate(
        [x.reshape(B * T, D), jnp.zeros((8, D), x.dtype)], axis=0)
    out_flat, mel16 = _upsample_sc(table, duration_target)
    out = out_flat.reshape(B, MAX_LEN, D)
    mel_len = mel16[:, 0]
    return out, duration_target, mel_len


# speedup vs baseline: 40.5268x; 6.4216x over previous
import jax
import jax.numpy as jnp
from jax import lax
from jax.experimental import pallas as pl
from jax.experimental.pallas import tpu as pltpu
from jax.experimental.pallas import tpu_sc as plsc

B, T, D = 8, 2048, 256
MAX_LEN = 14336
NC, NS, L = 2, 16, 16
BPC = B // NC
ROWS_PER_CORE = BPC * MAX_LEN
ROWS_PER_TILE = ROWS_PER_CORE // NS
CHUNK = 128
N_CHUNKS = ROWS_PER_TILE // CHUNK
WIN = 128
TCH = T // L
PCH = MAX_LEN // L
GROUPS = D // L


def _sc_body(x_hbm, dur_hbm, out_hbm, mel_hbm,
             dur_v, arr_v, gidx_sh, idx_v, x_win, out_buf, mel_v):
    c = lax.axis_index("c")
    s = lax.axis_index("s")

    @pl.when(s < BPC)
    def _build_indices():
        b = c * BPC + s
        pltpu.sync_copy(dur_hbm.at[b], dur_v)

        def zero_body(j, carry):
            arr_v[pl.ds(j * L, L)] = jnp.zeros((L,), jnp.int32)
            return carry
        lax.fori_loop(0, PCH, zero_body, jnp.int32(0))

        def scatter_body(i, carry):
            dv = dur_v[pl.ds(i * L, L)]
            cs = plsc.cumsum(dv) + carry
            start = cs - dv
            vals = lax.iota(jnp.int32, L) + i * L
            ok = (dv > 0) & (start < MAX_LEN)
            plsc.store_scatter(arr_v, [start], vals, mask=ok)
            return jnp.max(cs)
        total = lax.fori_loop(0, TCH, scatter_body, jnp.int32(0))

        def fill_body(j, carry):
            a = arr_v[pl.ds(j * L, L)]
            m = jnp.maximum(plsc.cummax(a), carry)
            pos = lax.iota(jnp.int32, L) + j * L
            arr_v[pl.ds(j * L, L)] = jnp.where(pos < total, m, -1)
            return jnp.max(m)
        lax.fori_loop(0, PCH, fill_body, jnp.int32(0))

        pltpu.sync_copy(arr_v, gidx_sh.at[pl.ds(s * MAX_LEN, MAX_LEN)])
        mel_v[...] = jnp.zeros((L,), jnp.int32) + total
        pltpu.sync_copy(mel_v, mel_hbm.at[b])

    plsc.subcore_barrier()

    core_row_base = c * ROWS_PER_CORE
    tile_off = s * ROWS_PER_TILE
    x_base = (c * BPC + s // (NS // BPC)) * T

    def chunk_body(k, carry):
        off = tile_off + k * CHUNK
        pltpu.sync_copy(gidx_sh.at[pl.ds(off, CHUNK)], idx_v)

        mv = idx_v[pl.ds(0, L)]
        for q in range(1, CHUNK // L):
            mv = jnp.maximum(mv, idx_v[pl.ds(q * L, L)])
        max_idx = jnp.max(mv)

        def window_pass(wb, first):
            src = pl.multiple_of(x_base + wb, 8)
            pltpu.sync_copy(x_hbm.at[pl.ds(src, WIN)], x_win)

            def frame_group(q, fcarry):
                iv = idx_v[pl.ds(q * L, L)]
                for lane in range(L):
                    ip = iv[lane]
                    lp = ip - wb
                    p = q * L + lane

                    @pl.when((lp >= 0) & (lp < WIN))
                    def _copy():
                        for g in range(GROUPS):
                            out_buf[p, pl.ds(g * L, L)] = \
                                x_win[lp, pl.ds(g * L, L)]

                    if first:
                        @pl.when(ip < 0)
                        def _zero():
                            zv = jnp.zeros((L,), jnp.float32)
                            for g in range(GROUPS):
                                out_buf[p, pl.ds(g * L, L)] = zv
                return fcarry
            lax.fori_loop(0, CHUNK // L, frame_group, jnp.int32(0))

        win0 = jnp.minimum(
            (jnp.maximum(idx_v[pl.ds(0, L)][0], 0) // 8) * 8, T - WIN)
        window_pass(win0, True)

        def wcond(wb):
            return wb + WIN <= max_idx

        def wbody(wb):
            nwb = jnp.minimum(wb + WIN, T - WIN)
            window_pass(nwb, False)
            return nwb
        lax.while_loop(wcond, wbody, win0)

        pltpu.sync_copy(out_buf, out_hbm.at[pl.ds(core_row_base + off, CHUNK)])
        return carry
    lax.fori_loop(0, N_CHUNKS, chunk_body, jnp.int32(0))


@jax.jit
def _upsample_sc(x_flat, duration):
    mesh = plsc.VectorSubcoreMesh(core_axis_name="c", subcore_axis_name="s")
    f = pl.kernel(
        _sc_body,
        mesh=mesh,
        compiler_params=pltpu.CompilerParams(needs_layout_passes=False),
        out_type=[
            jax.ShapeDtypeStruct((B * MAX_LEN, D), jnp.float32),
            jax.ShapeDtypeStruct((B, L), jnp.int32),
        ],
        scratch_types=[
            pltpu.VMEM((T,), jnp.int32),
            pltpu.VMEM((MAX_LEN,), jnp.int32),
            pltpu.VMEM_SHARED((BPC * MAX_LEN,), jnp.int32),
            pltpu.VMEM((CHUNK,), jnp.int32),
            pltpu.VMEM((WIN, D), jnp.float32),
            pltpu.VMEM((CHUNK, D), jnp.float32),
            pltpu.VMEM((L,), jnp.int32),
        ],
    )
    return f(x_flat, duration)


def kernel(x, log_duration_prediction, max_len, duration_target):
    del log_duration_prediction, max_len
    out_flat, mel16 = _upsample_sc(x.reshape(B * T, D), duration_target)
    out = out_flat.reshape(B, MAX_LEN, D)
    mel_len = mel16[:, 0]
    return out, duration_target, mel_len


# speedup vs baseline: 47.2050x; 1.1648x over previous
import jax
import jax.numpy as jnp
from jax import lax
from jax.experimental import pallas as pl
from jax.experimental.pallas import tpu as pltpu
from jax.experimental.pallas import tpu_sc as plsc

B, T, D = 8, 2048, 256
MAX_LEN = 14336
NC, NS, L = 2, 16, 16
BPC = B // NC
ROWS_PER_CORE = BPC * MAX_LEN
ROWS_PER_TILE = ROWS_PER_CORE // NS
CHUNK = 128
N_CHUNKS = ROWS_PER_TILE // CHUNK
WIN = 128
TCH = T // L
PCH = MAX_LEN // L
GROUPS = D // L


def _sc_body(x_hbm, dur_hbm, out_hbm, mel_hbm,
             dur_v, arr_v, gidx_sh, idx_all, x_win, out_buf0, out_buf1,
             mel_v, wsem0, wsem1):
    c = lax.axis_index("c")
    s = lax.axis_index("s")

    @pl.when(s < BPC)
    def _build_indices():
        b = c * BPC + s
        pltpu.sync_copy(dur_hbm.at[b], dur_v)

        def zero_body(j, carry):
            arr_v[pl.ds(j * L, L)] = jnp.zeros((L,), jnp.int32)
            return carry
        lax.fori_loop(0, PCH, zero_body, jnp.int32(0))

        def scatter_body(i, carry):
            dv = dur_v[pl.ds(i * L, L)]
            cs = plsc.cumsum(dv) + carry
            start = cs - dv
            vals = lax.iota(jnp.int32, L) + i * L
            ok = (dv > 0) & (start < MAX_LEN)
            plsc.store_scatter(arr_v, [start], vals, mask=ok)
            return jnp.max(cs)
        total = lax.fori_loop(0, TCH, scatter_body, jnp.int32(0))

        def fill_body(j, carry):
            a = arr_v[pl.ds(j * L, L)]
            m = jnp.maximum(plsc.cummax(a), carry)
            pos = lax.iota(jnp.int32, L) + j * L
            arr_v[pl.ds(j * L, L)] = jnp.where(pos < total, m, -1)
            return jnp.max(m)
        lax.fori_loop(0, PCH, fill_body, jnp.int32(0))

        pltpu.sync_copy(arr_v, gidx_sh.at[pl.ds(s * MAX_LEN, MAX_LEN)])
        mel_v[...] = jnp.zeros((L,), jnp.int32) + total
        pltpu.sync_copy(mel_v, mel_hbm.at[b])

    plsc.subcore_barrier()

    core_row_base = c * ROWS_PER_CORE
    tile_off = s * ROWS_PER_TILE
    x_base = (c * BPC + s // (NS // BPC)) * T

    pltpu.sync_copy(gidx_sh.at[pl.ds(tile_off, ROWS_PER_TILE)], idx_all)

    for g in range(GROUPS):
        x_win[WIN, pl.ds(g * L, L)] = jnp.zeros((L,), jnp.float32)

    out_bufs = (out_buf0, out_buf1)
    sems = (wsem0, wsem1)

    def process_chunk(k, out_buf):
        kbase = k * CHUNK
        mv = idx_all[pl.ds(kbase, L)]
        for q in range(1, CHUNK // L):
            mv = jnp.maximum(mv, idx_all[pl.ds(kbase + q * L, L)])
        max_idx = jnp.max(mv)

        def window_pass(wb):
            src = pl.multiple_of(x_base + wb, 8)
            pltpu.sync_copy(x_hbm.at[pl.ds(src, WIN)], x_win.at[pl.ds(0, WIN)])

            def frame_group(q, fcarry):
                iv = idx_all[pl.ds(kbase + q * L, L)]
                lpv = jnp.where(iv < 0, wb + WIN, iv) - wb
                for lane in range(L):
                    lp = lpv[lane]
                    p = q * L + lane

                    @pl.when((lp >= 0) & (lp <= WIN))
                    def _copy():
                        for g in range(GROUPS):
                            out_buf[p, pl.ds(g * L, L)] = \
                                x_win[lp, pl.ds(g * L, L)]
                return fcarry
            lax.fori_loop(0, CHUNK // L, frame_group, jnp.int32(0))

        win0 = jnp.minimum(
            (jnp.maximum(idx_all[pl.ds(kbase, L)][0], 0) // 8) * 8, T - WIN)
        window_pass(win0)

        def wcond(wb):
            return wb + WIN <= max_idx

        def wbody(wb):
            nwb = jnp.minimum(wb + WIN, T - WIN)
            window_pass(nwb)
            return nwb
        lax.while_loop(wcond, wbody, win0)

    def out_slice(k):
        return out_hbm.at[pl.ds(core_row_base + tile_off + k * CHUNK, CHUNK)]

    def pair_body(gp, carry):
        for bi in range(2):
            k = 2 * gp + bi

            @pl.when(gp > 0)
            def _drain():
                pltpu.make_async_copy(out_bufs[bi], out_slice(k), sems[bi]).wait()

            process_chunk(k, out_bufs[bi])
            pltpu.async_copy(out_bufs[bi], out_slice(k), sems[bi])
        return carry
    lax.fori_loop(0, N_CHUNKS // 2, pair_body, jnp.int32(0))

    for bi in range(2):
        pltpu.make_async_copy(out_bufs[bi], out_slice(0), sems[bi]).wait()


@jax.jit
def _upsample_sc(x_flat, duration):
    mesh = plsc.VectorSubcoreMesh(core_axis_name="c", subcore_axis_name="s")
    f = pl.kernel(
        _sc_body,
        mesh=mesh,
        compiler_params=pltpu.CompilerParams(needs_layout_passes=False),
        out_type=[
            jax.ShapeDtypeStruct((B * MAX_LEN, D), jnp.float32),
            jax.ShapeDtypeStruct((B, L), jnp.int32),
        ],
        scratch_types=[
            pltpu.VMEM((T,), jnp.int32),
            pltpu.VMEM((MAX_LEN,), jnp.int32),
            pltpu.VMEM_SHARED((BPC * MAX_LEN,), jnp.int32),
            pltpu.VMEM((ROWS_PER_TILE,), jnp.int32),
            pltpu.VMEM((WIN + 8, D), jnp.float32),
            pltpu.VMEM((CHUNK, D), jnp.float32),
            pltpu.VMEM((CHUNK, D), jnp.float32),
            pltpu.VMEM((L,), jnp.int32),
            pltpu.SemaphoreType.DMA,
            pltpu.SemaphoreType.DMA,
        ],
    )
    return f(x_flat, duration)


def kernel(x, log_duration_prediction, max_len, duration_target):
    del log_duration_prediction, max_len
    out_flat, mel16 = _upsample_sc(x.reshape(B * T, D), duration_target)
    out = out_flat.reshape(B, MAX_LEN, D)
    mel_len = mel16[:, 0]
    return out, duration_target, mel_len


# speedup vs baseline: 60.0586x; 1.2723x over previous
import jax
import jax.numpy as jnp
from jax import lax
from jax.experimental import pallas as pl
from jax.experimental.pallas import tpu as pltpu
from jax.experimental.pallas import tpu_sc as plsc

B, T, D = 8, 2048, 256
MAX_LEN = 14336
NC, NS, L = 2, 16, 16
BPC = B // NC
ROWS_PER_CORE = BPC * MAX_LEN
ROWS_PER_TILE = ROWS_PER_CORE // NS
CHUNK = 128
N_CHUNKS = ROWS_PER_TILE // CHUNK
WIN = 64
TCH = T // L
PCH = MAX_LEN // L
GROUPS = D // L


def _sc_body(x_hbm, dur_hbm, out_hbm, mel_hbm,
             dur_v, arr_v, gidx_sh, idx_all, x_win0, x_win1,
             out_buf0, out_buf1, mel_v, wsem0, wsem1, xsem0, xsem1):
    c = lax.axis_index("c")
    s = lax.axis_index("s")

    @pl.when(s < BPC)
    def _build_indices():
        b = c * BPC + s
        pltpu.sync_copy(dur_hbm.at[b], dur_v)

        def zero_body(j, carry):
            arr_v[pl.ds(j * L, L)] = jnp.zeros((L,), jnp.int32)
            return carry
        lax.fori_loop(0, PCH, zero_body, jnp.int32(0))

        def scatter_body(i, carry):
            dv = dur_v[pl.ds(i * L, L)]
            cs = plsc.cumsum(dv) + carry
            start = cs - dv
            vals = lax.iota(jnp.int32, L) + i * L
            ok = (dv > 0) & (start < MAX_LEN)
            plsc.store_scatter(arr_v, [start], vals, mask=ok)
            return jnp.max(cs)
        total = lax.fori_loop(0, TCH, scatter_body, jnp.int32(0))

        def fill_body(j, carry):
            a = arr_v[pl.ds(j * L, L)]
            m = jnp.maximum(plsc.cummax(a), carry)
            pos = lax.iota(jnp.int32, L) + j * L
            arr_v[pl.ds(j * L, L)] = jnp.where(pos < total, m, -1)
            return jnp.max(m)
        lax.fori_loop(0, PCH, fill_body, jnp.int32(0))

        pltpu.sync_copy(arr_v, gidx_sh.at[pl.ds(s * MAX_LEN, MAX_LEN)])
        mel_v[...] = jnp.zeros((L,), jnp.int32) + total
        pltpu.sync_copy(mel_v, mel_hbm.at[b])

    plsc.subcore_barrier()

    core_row_base = c * ROWS_PER_CORE
    tile_off = s * ROWS_PER_TILE
    x_base = (c * BPC + s // (NS // BPC)) * T

    pltpu.sync_copy(gidx_sh.at[pl.ds(tile_off, ROWS_PER_TILE)], idx_all)

    x_wins = (x_win0, x_win1)
    out_bufs = (out_buf0, out_buf1)
    wsems = (wsem0, wsem1)
    xsems = (xsem0, xsem1)

    for g in range(GROUPS):
        x_win0[WIN, pl.ds(g * L, L)] = jnp.zeros((L,), jnp.float32)
        x_win1[WIN, pl.ds(g * L, L)] = jnp.zeros((L,), jnp.float32)

    def win0_of(k):
        head = idx_all[pl.ds(k * CHUNK, L)][0]
        return jnp.minimum((jnp.maximum(head, 0) // 8) * 8, T - WIN)

    def win_src(wb):
        return x_hbm.at[pl.ds(pl.multiple_of(x_base + wb, 8), WIN)]

    def process_chunk(k, x_win, out_buf):
        kbase = k * CHUNK
        mv = idx_all[pl.ds(kbase, L)]
        for q in range(1, CHUNK // L):
            mv = jnp.maximum(mv, idx_all[pl.ds(kbase + q * L, L)])
        max_idx = jnp.max(mv)
        wb0 = win0_of(k)

        def frame_group0(q, fcarry):
            iv = idx_all[pl.ds(kbase + q * L, L)]
            lpv = jnp.minimum(jnp.where(iv < 0, WIN, iv - wb0), WIN)
            for lane in range(L):
                lp = lpv[lane]
                p = q * L + lane
                for g in range(GROUPS):
                    out_buf[p, pl.ds(g * L, L)] = x_win[lp, pl.ds(g * L, L)]
            return fcarry
        lax.fori_loop(0, CHUNK // L, frame_group0, jnp.int32(0))

        def extra_pass(wb):
            pltpu.sync_copy(win_src(wb), x_win.at[pl.ds(0, WIN)])

            def frame_group(q, fcarry):
                iv = idx_all[pl.ds(kbase + q * L, L)]
                lpv = jnp.minimum(jnp.where(iv < 0, wb + WIN, iv) - wb, WIN)
                for lane in range(L):
                    lp = lpv[lane]
                    p = q * L + lane

                    @pl.when(lp >= 0)
                    def _copy():
                        for g in range(GROUPS):
                            out_buf[p, pl.ds(g * L, L)] = \
                                x_win[lp, pl.ds(g * L, L)]
                return fcarry
            lax.fori_loop(0, CHUNK // L, frame_group, jnp.int32(0))

        def wcond(wb):
            return wb + WIN <= max_idx

        def wbody(wb):
            nwb = jnp.minimum(wb + WIN, T - WIN)
            extra_pass(nwb)
            return nwb
        lax.while_loop(wcond, wbody, wb0)

    def out_slice(k):
        return out_hbm.at[pl.ds(core_row_base + tile_off + k * CHUNK, CHUNK)]

    pltpu.async_copy(win_src(win0_of(0)), x_win0.at[pl.ds(0, WIN)], xsem0)

    def pair_body(gp, carry):
        for bi in range(2):
            k = 2 * gp + bi
            pltpu.make_async_copy(
                win_src(0), x_wins[bi].at[pl.ds(0, WIN)], xsems[bi]).wait()
            @pl.when(k + 1 < N_CHUNKS)
            def _prefetch():
                pltpu.async_copy(
                    win_src(win0_of(k + 1)),
                    x_wins[1 - bi].at[pl.ds(0, WIN)], xsems[1 - bi])

            @pl.when(gp > 0)
            def _drain():
                pltpu.make_async_copy(
                    out_bufs[bi], out_slice(k), wsems[bi]).wait()

            process_chunk(k, x_wins[bi], out_bufs[bi])
            pltpu.async_copy(out_bufs[bi], out_slice(k), wsems[bi])
        return carry
    lax.fori_loop(0, N_CHUNKS // 2, pair_body, jnp.int32(0))

    for bi in range(2):
        pltpu.make_async_copy(out_bufs[bi], out_slice(0), wsems[bi]).wait()


@jax.jit
def _upsample_sc(x_flat, duration):
    mesh = plsc.VectorSubcoreMesh(core_axis_name="c", subcore_axis_name="s")
    f = pl.kernel(
        _sc_body,
        mesh=mesh,
        compiler_params=pltpu.CompilerParams(needs_layout_passes=False),
        out_type=[
            jax.ShapeDtypeStruct((B * MAX_LEN, D), jnp.float32),
            jax.ShapeDtypeStruct((B, L), jnp.int32),
        ],
        scratch_types=[
            pltpu.VMEM((T,), jnp.int32),
            pltpu.VMEM((MAX_LEN,), jnp.int32),
            pltpu.VMEM_SHARED((BPC * MAX_LEN,), jnp.int32),
            pltpu.VMEM((ROWS_PER_TILE,), jnp.int32),
            pltpu.VMEM((WIN + 8, D), jnp.float32),
            pltpu.VMEM((WIN + 8, D), jnp.float32),
            pltpu.VMEM((CHUNK, D), jnp.float32),
            pltpu.VMEM((CHUNK, D), jnp.float32),
            pltpu.VMEM((L,), jnp.int32),
            pltpu.SemaphoreType.DMA,
            pltpu.SemaphoreType.DMA,
            pltpu.SemaphoreType.DMA,
            pltpu.SemaphoreType.DMA,
        ],
    )
    return f(x_flat, duration)


def kernel(x, log_duration_prediction, max_len, duration_target):
    del log_duration_prediction, max_len
    out_flat, mel16 = _upsample_sc(x.reshape(B * T, D), duration_target)
    out = out_flat.reshape(B, MAX_LEN, D)
    mel_len = mel16[:, 0]
    return out, duration_target, mel_len
